# baseline (speedup 1.0000x reference)
import functools
import jax
import jax.numpy as jnp
from jax import lax
from jax.experimental import pallas as pl
from jax.experimental.pallas import tpu as pltpu
from jax.experimental.pallas import tpu_sc as plsc

B = 2
S = 4096
D = 1024
NC, NS = 2, 16
NW = NC * NS
ROWS_W = S // NW
CH = 8
CHW = CH * D
NCHUNK = ROWS_W // CH
NBUF = 3

_mesh = plsc.VectorSubcoreMesh(core_axis_name="c", subcore_axis_name="s")


@functools.partial(
    pl.kernel,
    mesh=_mesh,
    out_type=jax.ShapeDtypeStruct((B * S * D,), jnp.float32),
    scratch_types=[
        pltpu.VMEM((NBUF * B * CHW,), jnp.float32),
        pltpu.VMEM((NBUF * CHW,), jnp.float32),
        pltpu.SemaphoreType.DMA,
        pltpu.SemaphoreType.DMA,
        pltpu.SemaphoreType.DMA,
        pltpu.SemaphoreType.DMA,
        pltpu.SemaphoreType.DMA,
        pltpu.SemaphoreType.DMA,
    ],
)
def _sc_add(x_hbm, pos_hbm, out_hbm, xbuf, pbuf, si0, si1, si2, so0, so1, so2):
    sin = (si0, si1, si2)
    sout = (so0, so1, so2)
    wid = lax.axis_index("s") * NC + lax.axis_index("c")
    s_base = wid * ROWS_W * D

    def issue_in(g):
        slot = g % NBUF
        off = s_base + g * CHW
        h = [pltpu.async_copy(pos_hbm.at[pl.ds(off, CHW)],
                              pbuf.at[pl.ds(slot * CHW, CHW)], sin[slot])]
        for b in range(B):
            h.append(pltpu.async_copy(
                x_hbm.at[pl.ds(b * S * D + off, CHW)],
                xbuf.at[pl.ds((slot * B + b) * CHW, CHW)], sin[slot]))
        return h

    def issue_out(g):
        slot = g % NBUF
        off = s_base + g * CHW
        h = []
        for b in range(B):
            h.append(pltpu.async_copy(
                xbuf.at[pl.ds((slot * B + b) * CHW, CHW)],
                out_hbm.at[pl.ds(b * S * D + off, CHW)], sout[slot]))
        return h

    def compute(g):
        slot = g % NBUF

        def body(i, _):
            o = i * 16
            p = pbuf[pl.ds(slot * CHW + o, 16)]
            for b in range(B):
                plsc.addupdate(xbuf.at[pl.ds((slot * B + b) * CHW + o, 16)], p)
            return 0

        lax.fori_loop(0, CHW // 16, body, 0)

    hin = {0: issue_in(0), 1: issue_in(1)}
    hout = {}
    for g in range(NCHUNK):
        for h in hin.pop(g):
            h.wait()
        compute(g)
        hout[g] = issue_out(g)
        if g - 1 in hout:
            for h in hout.pop(g - 1):
                h.wait()
        if g + 2 < NCHUNK:
            hin[g + 2] = issue_in(g + 2)
    for g in list(hout):
        for h in hout.pop(g):
            h.wait()


def kernel(x, pos_table):
    xf = x.reshape(-1)
    pf = pos_table.reshape(-1)
    out = _sc_add(xf, pf)
    return out.reshape(x.shape)


# speedup vs baseline: 2.4082x; 2.4082x over previous
import functools
import jax
import jax.numpy as jnp
from jax import lax
from jax.experimental import pallas as pl
from jax.experimental.pallas import tpu as pltpu
from jax.experimental.pallas import tpu_sc as plsc

B = 2
S = 4096
D = 1024
NC, NS = 2, 16
NW = NC * NS
ROWS_W = S // NW
CH = 8
NCHUNK = ROWS_W // CH
NBUF = 3

_mesh = plsc.VectorSubcoreMesh(core_axis_name="c", subcore_axis_name="s")


@functools.partial(
    pl.kernel,
    mesh=_mesh,
    out_type=jax.ShapeDtypeStruct((B * S, D), jnp.float32),
    scratch_types=[
        pltpu.VMEM((NBUF, B, CH, D), jnp.float32),
        pltpu.VMEM((NBUF, CH, D), jnp.float32),
        pltpu.SemaphoreType.DMA,
        pltpu.SemaphoreType.DMA,
        pltpu.SemaphoreType.DMA,
        pltpu.SemaphoreType.DMA,
        pltpu.SemaphoreType.DMA,
        pltpu.SemaphoreType.DMA,
    ],
)
def _sc_add(x_hbm, pos_hbm, out_hbm, xbuf, pbuf, si0, si1, si2, so0, so1, so2):
    sin = (si0, si1, si2)
    sout = (so0, so1, so2)
    wid = lax.axis_index("s") * NC + lax.axis_index("c")
    row_base = wid * ROWS_W

    def issue_in(g):
        slot = g % NBUF
        r0 = row_base + g * CH
        h = [pltpu.async_copy(pos_hbm.at[pl.ds(r0, CH), :],
                              pbuf.at[slot], sin[slot])]
        for b in range(B):
            h.append(pltpu.async_copy(
                x_hbm.at[pl.ds(b * S + r0, CH), :],
                xbuf.at[slot, b], sin[slot]))
        return h

    def issue_out(g):
        slot = g % NBUF
        r0 = row_base + g * CH
        h = []
        for b in range(B):
            h.append(pltpu.async_copy(
                xbuf.at[slot, b],
                out_hbm.at[pl.ds(b * S + r0, CH), :], sout[slot]))
        return h

    def compute(g):
        slot = g % NBUF

        def body(i, _):
            c = i * 16
            for r in range(CH):
                p = pbuf[slot, r, pl.ds(c, 16)]
                for b in range(B):
                    plsc.addupdate(xbuf.at[slot, b, r, pl.ds(c, 16)], p)
            return 0

        lax.fori_loop(0, D // 16, body, 0)

    hin = {0: issue_in(0), 1: issue_in(1)}
    hout = {}
    for g in range(NCHUNK):
        for h in hin.pop(g):
            h.wait()
        compute(g)
        hout[g] = issue_out(g)
        if g - 1 in hout:
            for h in hout.pop(g - 1):
                h.wait()
        if g + 2 < NCHUNK:
            hin[g + 2] = issue_in(g + 2)
    for g in list(hout):
        for h in hout.pop(g):
            h.wait()


def kernel(x, pos_table):
    xf = x.reshape(B * S, D)
    out = _sc_add(xf, pos_table)
    return out.reshape(x.shape)
